# baseline (speedup 1.0000x reference)
import functools

import jax
import jax.numpy as jnp
from jax import lax
from jax.experimental import pallas as pl
from jax.experimental.pallas import tpu as pltpu
from jax.experimental.pallas import tpu_sc as plsc

N_ROWS = 1_000_000
K = 64
B = 16384

_L = 16
_UNROLL = 16
_DCH = 2000
_CHUNK = 128


def _build_depad():
    def body(i_ref, o_ref):
        x = i_ref[...].reshape(_DCH // 2, 2, K)
        o_ref[:, :K] = x[:, 0, :]
        o_ref[:, K:] = x[:, 1, :]

    return pl.pallas_call(
        body,
        grid=(N_ROWS // _DCH,),
        in_specs=[pl.BlockSpec((_DCH, K), lambda i: (i, 0))],
        out_specs=pl.BlockSpec((_DCH // 2, 2 * K), lambda i: (i, 0)),
        out_shape=jax.ShapeDtypeStruct((N_ROWS // 2, 2 * K), jnp.float32),
    )


def _build_rowgather():
    info = plsc.get_sparse_core_info()
    nc, ns = info.num_cores, info.num_subcores
    nw = nc * ns
    b_per_w = B // nw
    mesh = plsc.VectorSubcoreMesh(core_axis_name="c", subcore_axis_name="s")

    @functools.partial(
        pl.kernel,
        mesh=mesh,
        out_type=jax.ShapeDtypeStruct((B, K), jnp.float32),
        scratch_types=[
            pltpu.VMEM_SHARED((ns, b_per_w), jnp.int32),
            pltpu.SMEM((b_per_w,), jnp.int32),
            pltpu.VMEM((b_per_w, K), jnp.float32),
            pltpu.SemaphoreType.DMA,
        ],
        compiler_params=pltpu.CompilerParams(needs_layout_passes=False),
    )
    def k(idx_hbm, tbl_hbm, out_hbm, idx_sh, idx_s, rows_v, sem):
        cid = lax.axis_index("c")
        sid = lax.axis_index("s")
        wid = sid * nc + cid
        base = wid * b_per_w
        pltpu.sync_copy(idx_hbm.at[pl.ds(base, b_per_w)], idx_sh.at[sid])
        pltpu.sync_copy(idx_sh.at[sid], idx_s)

        def fire(g, _):
            for j in range(_UNROLL):
                i = g * _UNROLL + j
                pltpu.async_copy(tbl_hbm.at[idx_s[i]], rows_v.at[i], sem)
            return _

        lax.fori_loop(0, b_per_w // _UNROLL, fire, None)

        def drain(i, _):
            pltpu.make_async_copy(tbl_hbm.at[0], rows_v.at[0], sem).wait()
            return _

        lax.fori_loop(0, b_per_w, drain, None)
        pltpu.sync_copy(rows_v, out_hbm.at[pl.ds(base, b_per_w)])

    return k


def _build_densegather():
    info = plsc.get_sparse_core_info()
    nc, ns = info.num_cores, info.num_subcores
    nw = nc * ns
    b_per_w = B // nw
    n_chunks = b_per_w // _CHUNK
    n_groups = b_per_w // _L
    mesh = plsc.VectorSubcoreMesh(core_axis_name="c", subcore_axis_name="s")

    @functools.partial(
        pl.kernel,
        mesh=mesh,
        out_type=jax.ShapeDtypeStruct((B, K), jnp.float32),
        scratch_types=[
            pltpu.VMEM((b_per_w,), jnp.int32),
            pltpu.VMEM((b_per_w,), jnp.int32),
            pltpu.VMEM((b_per_w // 2, 2 * K), jnp.float32),
            pltpu.VMEM((b_per_w // 2, K), jnp.float32),
            pltpu.SemaphoreType.DMA,
        ],
        compiler_params=pltpu.CompilerParams(needs_layout_passes=False),
    )
    def k(idx_hbm, tbl_hbm, out_hbm, idx_v, g_v, buf, out_v, sem):
        cid = lax.axis_index("c")
        sid = lax.axis_index("s")
        wid = sid * nc + cid
        base = wid * b_per_w
        pltpu.sync_copy(idx_hbm.at[pl.ds(base, b_per_w)], idx_v)
        for i in range(n_groups):
            v = idx_v[pl.ds(i * _L, _L)]
            g_v[pl.ds(i * _L, _L)] = v >> 1

        iota = lax.iota(jnp.int32, _L)
        half = b_per_w // 2

        for r in range(2):
            ho = r * half
            copies = []
            for j in range(n_chunks // 2):
                o = j * _CHUNK
                copies.append(
                    pltpu.async_copy(
                        tbl_hbm.at[g_v.at[pl.ds(ho + o, _CHUNK)]],
                        buf.at[pl.ds(o, _CHUNK)],
                        sem,
                    )
                )
            for c in copies:
                c.wait()

            def extract(g, _):
                v = idx_v[pl.ds(ho + g * _L, _L)]
                col0 = (v & 1) << 6
                row = iota + g * _L
                for j in range(K):
                    x = plsc.load_gather(buf, [row, col0 + j])
                    plsc.store_scatter(
                        out_v, [row, jnp.full((_L,), j, jnp.int32)], x
                    )
                return _

            lax.fori_loop(0, n_groups // 2, extract, None)
            pltpu.sync_copy(out_v, out_hbm.at[pl.ds(base + ho, half)])

    return k


_depad = _build_depad()
_rowgather = _build_rowgather()
_densegather = _build_densegather()


def kernel(indices, mu, log_sigma):
    idx = indices.astype(jnp.int32)
    ls_out = _rowgather(idx, log_sigma)
    mu_dense = _depad(mu)
    mu_out = _densegather(idx, mu_dense)
    return (mu_out, ls_out)


# speedup vs baseline: 1.7855x; 1.7855x over previous
import functools

import jax
import jax.numpy as jnp
from jax import lax
from jax.experimental import pallas as pl
from jax.experimental.pallas import tpu as pltpu
from jax.experimental.pallas import tpu_sc as plsc

N_ROWS = 1_000_000
K = 64
B = 16384

_L = 16
_UNROLL = 16
_DCH = 2000
_CHUNK = 128


def _build_depad():
    def body(i_ref, o_ref):
        x = i_ref[...].reshape(_DCH // 2, 2, K)
        o_ref[:, :K] = x[:, 0, :]
        o_ref[:, K:] = x[:, 1, :]

    return pl.pallas_call(
        body,
        grid=(N_ROWS // _DCH,),
        in_specs=[pl.BlockSpec((_DCH, K), lambda i: (i, 0))],
        out_specs=pl.BlockSpec((_DCH // 2, 2 * K), lambda i: (i, 0)),
        out_shape=jax.ShapeDtypeStruct((N_ROWS // 2, 2 * K), jnp.float32),
    )


def _build_rowgather():
    info = plsc.get_sparse_core_info()
    nc, ns = info.num_cores, info.num_subcores
    nw = nc * ns
    b_per_w = B // nw
    mesh = plsc.VectorSubcoreMesh(core_axis_name="c", subcore_axis_name="s")

    @functools.partial(
        pl.kernel,
        mesh=mesh,
        out_type=jax.ShapeDtypeStruct((B, K), jnp.float32),
        scratch_types=[
            pltpu.VMEM_SHARED((ns, b_per_w), jnp.int32),
            pltpu.SMEM((b_per_w,), jnp.int32),
            pltpu.VMEM((b_per_w, K), jnp.float32),
            pltpu.SemaphoreType.DMA,
        ],
        compiler_params=pltpu.CompilerParams(needs_layout_passes=False),
    )
    def k(idx_hbm, tbl_hbm, out_hbm, idx_sh, idx_s, rows_v, sem):
        cid = lax.axis_index("c")
        sid = lax.axis_index("s")
        wid = sid * nc + cid
        base = wid * b_per_w
        pltpu.sync_copy(idx_hbm.at[pl.ds(base, b_per_w)], idx_sh.at[sid])
        pltpu.sync_copy(idx_sh.at[sid], idx_s)

        def fire(g, _):
            for j in range(_UNROLL):
                i = g * _UNROLL + j
                pltpu.async_copy(tbl_hbm.at[idx_s[i]], rows_v.at[i], sem)
            return _

        lax.fori_loop(0, b_per_w // _UNROLL, fire, None)

        def drain(i, _):
            pltpu.make_async_copy(tbl_hbm.at[0], rows_v.at[0], sem).wait()
            return _

        lax.fori_loop(0, b_per_w, drain, None)
        pltpu.sync_copy(rows_v, out_hbm.at[pl.ds(base, b_per_w)])

    return k


def _build_densegather():
    info = plsc.get_sparse_core_info()
    nc, ns = info.num_cores, info.num_subcores
    nw = nc * ns
    b_per_w = B // nw
    n_chunks = b_per_w // _CHUNK
    n_groups = b_per_w // _L
    mesh = plsc.VectorSubcoreMesh(core_axis_name="c", subcore_axis_name="s")

    @functools.partial(
        pl.kernel,
        mesh=mesh,
        out_type=jax.ShapeDtypeStruct((B, K), jnp.float32),
        scratch_types=[
            pltpu.VMEM((b_per_w,), jnp.int32),
            pltpu.VMEM((b_per_w,), jnp.int32),
            pltpu.VMEM((b_per_w // 2, 2 * K), jnp.float32),
            pltpu.VMEM((b_per_w // 2, K), jnp.float32),
            pltpu.SemaphoreType.DMA,
        ],
        compiler_params=pltpu.CompilerParams(needs_layout_passes=False),
    )
    def k(idx_hbm, tbl_hbm, out_hbm, idx_v, g_v, buf, out_v, sem):
        cid = lax.axis_index("c")
        sid = lax.axis_index("s")
        wid = sid * nc + cid
        base = wid * b_per_w
        pltpu.sync_copy(idx_hbm.at[pl.ds(base, b_per_w)], idx_v)
        for i in range(n_groups):
            v = idx_v[pl.ds(i * _L, _L)]
            g_v[pl.ds(i * _L, _L)] = v >> 1

        iota = lax.iota(jnp.int32, _L)
        half = b_per_w // 2

        for r in range(2):
            ho = r * half
            copies = []
            for j in range(n_chunks // 2):
                o = j * _CHUNK
                copies.append(
                    pltpu.async_copy(
                        tbl_hbm.at[g_v.at[pl.ds(ho + o, _CHUNK)]],
                        buf.at[pl.ds(o, _CHUNK)],
                        sem,
                    )
                )
            for c in copies:
                c.wait()

            def extract(g, _):
                v = idx_v[pl.ds(ho + g * _L, _L)]
                col0 = (v & 1) << 6
                row = iota + g * _L
                for j in range(K):
                    x = plsc.load_gather(buf, [row, col0 + j])
                    plsc.store_scatter(
                        out_v, [row, jnp.full((_L,), j, jnp.int32)], x
                    )
                return _

            lax.fori_loop(0, n_groups // 2, extract, None)
            pltpu.sync_copy(out_v, out_hbm.at[pl.ds(base + ho, half)])

    return k


_depad = _build_depad()
_rowgather = _build_rowgather()
_densegather = _build_densegather()


def kernel(indices, mu, log_sigma):
    idx = indices.astype(jnp.int32)
    ls_out = _rowgather(idx, log_sigma)
    mu_out = _rowgather(idx, mu)
    return (mu_out, ls_out)
